# baseline (speedup 1.0000x reference)
import jax
import jax.numpy as jnp
from jax.experimental import pallas as pl
from jax.experimental.pallas import tpu as pltpu

_TILE = 4096
_NSLOT = 4


def _body(x_hbm, cw_ref, cb_ref, m1_ref, b1_ref, m2_ref, b2_ref, o_hbm,
          xbuf, obuf, in_sem, out_sem):
    i = pl.program_id(0)
    n = pl.num_programs(0)

    @pl.when(i == 0)
    def _():
        for k in range(_NSLOT - 1):
            @pl.when(k < n)
            def _():
                pltpu.make_async_copy(
                    x_hbm.at[pl.ds(k * _TILE, _TILE), :], xbuf.at[k],
                    in_sem.at[k],
                ).start()

    nxt = i + _NSLOT - 1
    nslot = jax.lax.rem(nxt, _NSLOT)

    @pl.when(nxt < n)
    def _():
        pltpu.make_async_copy(
            x_hbm.at[pl.ds(nxt * _TILE, _TILE), :], xbuf.at[nslot],
            in_sem.at[nslot],
        ).start()

    slot = jax.lax.rem(i, _NSLOT)
    pltpu.make_async_copy(
        x_hbm.at[pl.ds(i * _TILE, _TILE), :], xbuf.at[slot], in_sem.at[slot]
    ).wait()

    @pl.when(i >= _NSLOT)
    def _():
        pltpu.make_async_copy(
            obuf.at[slot], o_hbm.at[pl.ds((i - _NSLOT) * _TILE, _TILE), :],
            out_sem.at[slot],
        ).wait()

    for c in range(2):
        rows = pl.ds(c * (_TILE // 2), _TILE // 2)
        a = jnp.maximum(xbuf[slot, rows, :] * cw_ref[...] + cb_ref[...], 0.0)
        t = jnp.dot(a, m1_ref[...], preferred_element_type=jnp.float32)
        t = jnp.maximum(t + b1_ref[...], 0.0)
        o = jnp.dot(t, m2_ref[...], preferred_element_type=jnp.float32)
        obuf[slot, rows, :] = jnp.maximum(o + b2_ref[...], 0.0)

    pltpu.make_async_copy(
        obuf.at[slot], o_hbm.at[pl.ds(i * _TILE, _TILE), :], out_sem.at[slot]
    ).start()

    @pl.when(i == n - 1)
    def _():
        for k in range(_NSLOT):
            j = i - (_NSLOT - 1) + k

            @pl.when(j >= 0)
            def _():
                jc = jnp.maximum(j, 0)
                pltpu.make_async_copy(
                    obuf.at[jax.lax.rem(jc, _NSLOT)],
                    o_hbm.at[pl.ds(jc * _TILE, _TILE), :],
                    out_sem.at[jax.lax.rem(jc, _NSLOT)],
                ).wait()


def kernel(x, conv_w, conv_b, W0, b0, W1, b1, W2, b2):
    B, I = x.shape
    D = W2.shape[0]
    m1 = jnp.zeros((I, 16), x.dtype).at[::8, :].set(W1[::16, :16].T)
    b1s = b1[::16].reshape(1, 16)
    m2 = W2[:, :16].T
    cw = conv_w.reshape(1, I)
    cb = conv_b.reshape(1, I)

    grid = (B // _TILE,)
    return pl.pallas_call(
        _body,
        grid=grid,
        in_specs=[
            pl.BlockSpec(memory_space=pl.ANY),
            pl.BlockSpec((1, I), lambda i: (0, 0)),
            pl.BlockSpec((1, I), lambda i: (0, 0)),
            pl.BlockSpec((I, 16), lambda i: (0, 0)),
            pl.BlockSpec((1, 16), lambda i: (0, 0)),
            pl.BlockSpec((16, D), lambda i: (0, 0)),
            pl.BlockSpec((1, D), lambda i: (0, 0)),
        ],
        out_specs=pl.BlockSpec(memory_space=pl.ANY),
        out_shape=jax.ShapeDtypeStruct((B, D), x.dtype),
        scratch_shapes=[
            pltpu.VMEM((_NSLOT, _TILE, I), jnp.float32),
            pltpu.VMEM((_NSLOT, _TILE, D), jnp.float32),
            pltpu.SemaphoreType.DMA((_NSLOT,)),
            pltpu.SemaphoreType.DMA((_NSLOT,)),
        ],
        compiler_params=pltpu.CompilerParams(
            dimension_semantics=("arbitrary",),
        ),
    )(x, cw, cb, m1, b1s, m2, b2.reshape(1, D))


# speedup vs baseline: 1.0335x; 1.0335x over previous
import jax
import jax.numpy as jnp
from jax.experimental import pallas as pl
from jax.experimental.pallas import tpu as pltpu

_TILE = 4096
_NSLOT = 4


def _body(x_hbm, cw_ref, cb_ref, m1_ref, b1_ref, m2_ref, b2_ref, o_hbm,
          xbuf, obuf, in_sem, out_sem):
    i = pl.program_id(0)
    n = pl.num_programs(0)

    @pl.when(i == 0)
    def _():
        for k in range(_NSLOT - 1):
            @pl.when(k < n)
            def _():
                pltpu.make_async_copy(
                    x_hbm.at[pl.ds(k * _TILE, _TILE), :], xbuf.at[k],
                    in_sem.at[k],
                ).start()

    nxt = i + _NSLOT - 1
    nslot = jax.lax.rem(nxt, _NSLOT)

    @pl.when(nxt < n)
    def _():
        pltpu.make_async_copy(
            x_hbm.at[pl.ds(nxt * _TILE, _TILE), :], xbuf.at[nslot],
            in_sem.at[nslot],
        ).start()

    slot = jax.lax.rem(i, _NSLOT)
    pltpu.make_async_copy(
        x_hbm.at[pl.ds(i * _TILE, _TILE), :], xbuf.at[slot], in_sem.at[slot]
    ).wait()

    @pl.when(i >= _NSLOT)
    def _():
        pltpu.make_async_copy(
            obuf.at[slot], o_hbm.at[pl.ds((i - _NSLOT) * _TILE, _TILE), :],
            out_sem.at[slot],
        ).wait()

    a = jnp.maximum(xbuf[slot] * cw_ref[...] + cb_ref[...], 0.0)
    t = jnp.dot(a, m1_ref[...], preferred_element_type=jnp.float32)
    t = jnp.maximum(t + b1_ref[...], 0.0)
    o = jnp.dot(t, m2_ref[...], preferred_element_type=jnp.float32)
    obuf[slot] = jnp.maximum(o + b2_ref[...], 0.0)

    pltpu.make_async_copy(
        obuf.at[slot], o_hbm.at[pl.ds(i * _TILE, _TILE), :], out_sem.at[slot]
    ).start()

    @pl.when(i == n - 1)
    def _():
        for k in range(_NSLOT):
            j = i - (_NSLOT - 1) + k

            @pl.when(j >= 0)
            def _():
                jc = jnp.maximum(j, 0)
                pltpu.make_async_copy(
                    obuf.at[jax.lax.rem(jc, _NSLOT)],
                    o_hbm.at[pl.ds(jc * _TILE, _TILE), :],
                    out_sem.at[jax.lax.rem(jc, _NSLOT)],
                ).wait()


def kernel(x, conv_w, conv_b, W0, b0, W1, b1, W2, b2):
    B, I = x.shape
    D = W2.shape[0]
    m1 = jnp.zeros((I, 16), x.dtype).at[::8, :].set(W1[::16, :16].T)
    b1s = b1[::16].reshape(1, 16)
    m2 = W2[:, :16].T
    cw = conv_w.reshape(1, I)
    cb = conv_b.reshape(1, I)

    grid = (B // _TILE,)
    return pl.pallas_call(
        _body,
        grid=grid,
        in_specs=[
            pl.BlockSpec(memory_space=pl.ANY),
            pl.BlockSpec((1, I), lambda i: (0, 0)),
            pl.BlockSpec((1, I), lambda i: (0, 0)),
            pl.BlockSpec((I, 16), lambda i: (0, 0)),
            pl.BlockSpec((1, 16), lambda i: (0, 0)),
            pl.BlockSpec((16, D), lambda i: (0, 0)),
            pl.BlockSpec((1, D), lambda i: (0, 0)),
        ],
        out_specs=pl.BlockSpec(memory_space=pl.ANY),
        out_shape=jax.ShapeDtypeStruct((B, D), x.dtype),
        scratch_shapes=[
            pltpu.VMEM((_NSLOT, _TILE, I), jnp.float32),
            pltpu.VMEM((_NSLOT, _TILE, D), jnp.float32),
            pltpu.SemaphoreType.DMA((_NSLOT,)),
            pltpu.SemaphoreType.DMA((_NSLOT,)),
        ],
        compiler_params=pltpu.CompilerParams(
            dimension_semantics=("arbitrary",),
        ),
    )(x, cw, cb, m1, b1s, m2, b2.reshape(1, D))
